# baseline (speedup 1.0000x reference)
import jax
import jax.numpy as jnp
from jax.experimental import pallas as pl
from jax.experimental.pallas import tpu as pltpu

_NEG_SLOPE = 0.2
_EPS = 1e-16


def _dense1_body(x_ref, batch_ref, se_ref, w1_ref, asw_ref, adw_ref,
                 h_ref, s_ref, d_ref):
    R = x_ref.shape[0]
    B = se_ref.shape[0]
    H, HID = asw_ref.shape
    onehot = (batch_ref[...] ==
              jax.lax.broadcasted_iota(jnp.int32, (R, B), 1)).astype(jnp.float32)
    cond = jnp.dot(onehot, se_ref[...], preferred_element_type=jnp.float32)
    hcat = jnp.concatenate([x_ref[...], cond], axis=1)
    for k in range(H):
        hk = jnp.dot(hcat, w1_ref[:, k * HID:(k + 1) * HID],
                     preferred_element_type=jnp.float32)
        h_ref[k] = hk
        s_ref[k] = jnp.sum(hk * asw_ref[k, :][None, :], axis=1, keepdims=True)
        d_ref[k] = jnp.sum(hk * adw_ref[k, :][None, :], axis=1, keepdims=True)


def _edge_body(src_ref, dst_ref, asrc_ref, adst_ref, h_ref, out_ref, den_ref):
    j = pl.program_id(1)

    @pl.when(j == 0)
    def _():
        out_ref[...] = jnp.zeros_like(out_ref)
        den_ref[...] = jnp.zeros_like(den_ref)

    CH = src_ref.shape[0]

    def body(e, carry):
        s = src_ref[e]
        d = dst_ref[e]
        logit = asrc_ref[:, pl.ds(s, 1)] + adst_ref[:, pl.ds(d, 1)]
        w = jnp.exp(jnp.where(logit >= 0, logit, _NEG_SLOPE * logit))
        den_ref[:, pl.ds(d, 1)] = den_ref[:, pl.ds(d, 1)] + w
        out_ref[:, pl.ds(d, 1)] = (out_ref[:, pl.ds(d, 1)]
                                   + h_ref[:, pl.ds(s, 1)] * w)
        return carry

    jax.lax.fori_loop(0, CH, body, 0)


def _edge_pass(src, dst, asrc, adst, h, CH):
    EP = src.shape[0]
    H, NP, C = h.shape
    return pl.pallas_call(
        _edge_body,
        grid=(H, EP // CH),
        in_specs=[
            pl.BlockSpec((CH,), lambda k, j: (j,), memory_space=pltpu.SMEM),
            pl.BlockSpec((CH,), lambda k, j: (j,), memory_space=pltpu.SMEM),
            pl.BlockSpec((1, NP, 1), lambda k, j: (k, 0, 0)),
            pl.BlockSpec((1, NP, 1), lambda k, j: (k, 0, 0)),
            pl.BlockSpec((1, NP, C), lambda k, j: (k, 0, 0)),
        ],
        out_specs=[
            pl.BlockSpec((1, NP, C), lambda k, j: (k, 0, 0)),
            pl.BlockSpec((1, NP, 1), lambda k, j: (k, 0, 0)),
        ],
        out_shape=[
            jax.ShapeDtypeStruct((H, NP, C), jnp.float32),
            jax.ShapeDtypeStruct((H, NP, 1), jnp.float32),
        ],
    )(src, dst, asrc, adst, h)


def _dense2_body(out1_ref, den1_ref, b1_ref, w2_ref, asw_ref, adw_ref,
                 h2_ref, s2_ref, d2_ref):
    H = b1_ref.shape[0]
    x2 = jnp.maximum(
        out1_ref[...] / (den1_ref[...] + _EPS) + b1_ref[...][:, None, :], 0.0)
    acc = None
    for k in range(H):
        p = jnp.dot(x2[k], w2_ref[k], preferred_element_type=jnp.float32)
        acc = p if acc is None else acc + p
    h2_ref[0] = acc
    s2_ref[0] = jnp.sum(acc * asw_ref[...], axis=1, keepdims=True)
    d2_ref[0] = jnp.sum(acc * adw_ref[...], axis=1, keepdims=True)


def _final_body(out2_ref, den2_ref, b2_ref, wo_ref, bo_ref, y_ref):
    v = jnp.maximum(
        out2_ref[0] / (den2_ref[0] + _EPS) + b2_ref[...], 0.0)
    y_ref[...] = jnp.dot(v, wo_ref[...],
                         preferred_element_type=jnp.float32) + bo_ref[...]


def kernel(x, edge_index, substring_embed, batch,
           W1, a_src1, a_dst1, b1, W2, a_src2, a_dst2, b2, Wo, bo):
    N, IN = x.shape
    B, COND = substring_embed.shape
    H, HID = a_src1.shape
    E = edge_index.shape[1]

    R = 256
    NP = ((N + 1 + R - 1) // R) * R
    CH = 8192
    EP0 = E + N
    EP = ((EP0 + CH - 1) // CH) * CH

    loop = jnp.arange(N, dtype=jnp.int32)
    src = jnp.concatenate([edge_index[0].astype(jnp.int32), loop,
                           jnp.full((EP - EP0,), N, jnp.int32)])
    dst = jnp.concatenate([edge_index[1].astype(jnp.int32), loop,
                           jnp.full((EP - EP0,), N, jnp.int32)])

    xp = jnp.pad(x, ((0, NP - N), (0, 0)))
    batchp = jnp.pad(batch.astype(jnp.int32), (0, NP - N))[:, None]

    grid = (NP // R,)
    h1, s1, d1 = pl.pallas_call(
        _dense1_body,
        grid=grid,
        in_specs=[
            pl.BlockSpec((R, IN), lambda i: (i, 0)),
            pl.BlockSpec((R, 1), lambda i: (i, 0)),
            pl.BlockSpec((B, COND), lambda i: (0, 0)),
            pl.BlockSpec((IN + COND, H * HID), lambda i: (0, 0)),
            pl.BlockSpec((H, HID), lambda i: (0, 0)),
            pl.BlockSpec((H, HID), lambda i: (0, 0)),
        ],
        out_specs=[
            pl.BlockSpec((H, R, HID), lambda i: (0, i, 0)),
            pl.BlockSpec((H, R, 1), lambda i: (0, i, 0)),
            pl.BlockSpec((H, R, 1), lambda i: (0, i, 0)),
        ],
        out_shape=[
            jax.ShapeDtypeStruct((H, NP, HID), jnp.float32),
            jax.ShapeDtypeStruct((H, NP, 1), jnp.float32),
            jax.ShapeDtypeStruct((H, NP, 1), jnp.float32),
        ],
    )(xp, batchp, substring_embed, W1, a_src1, a_dst1)

    out1, den1 = _edge_pass(src, dst, s1, d1, h1, CH)

    h2, s2, d2 = pl.pallas_call(
        _dense2_body,
        grid=grid,
        in_specs=[
            pl.BlockSpec((H, R, HID), lambda i: (0, i, 0)),
            pl.BlockSpec((H, R, 1), lambda i: (0, i, 0)),
            pl.BlockSpec((H, HID), lambda i: (0, 0)),
            pl.BlockSpec((H, HID, HID), lambda i: (0, 0, 0)),
            pl.BlockSpec((1, HID), lambda i: (0, 0)),
            pl.BlockSpec((1, HID), lambda i: (0, 0)),
        ],
        out_specs=[
            pl.BlockSpec((1, R, HID), lambda i: (0, i, 0)),
            pl.BlockSpec((1, R, 1), lambda i: (0, i, 0)),
            pl.BlockSpec((1, R, 1), lambda i: (0, i, 0)),
        ],
        out_shape=[
            jax.ShapeDtypeStruct((1, NP, HID), jnp.float32),
            jax.ShapeDtypeStruct((1, NP, 1), jnp.float32),
            jax.ShapeDtypeStruct((1, NP, 1), jnp.float32),
        ],
    )(out1, den1, b1.reshape(H, HID), W2.reshape(H, HID, HID),
      a_src2, a_dst2)

    out2, den2 = _edge_pass(src, dst, s2, d2, h2, CH)

    y = pl.pallas_call(
        _final_body,
        grid=grid,
        in_specs=[
            pl.BlockSpec((1, R, HID), lambda i: (0, i, 0)),
            pl.BlockSpec((1, R, 1), lambda i: (0, i, 0)),
            pl.BlockSpec((1, HID), lambda i: (0, 0)),
            pl.BlockSpec((HID, 1), lambda i: (0, 0)),
            pl.BlockSpec((1, 1), lambda i: (0, 0)),
        ],
        out_specs=pl.BlockSpec((R, 1), lambda i: (i, 0)),
        out_shape=jax.ShapeDtypeStruct((NP, 1), jnp.float32),
    )(out2, den2, b2.reshape(1, HID), Wo, bo.reshape(1, 1))

    return y[:N, 0]
